# baseline (speedup 1.0000x reference)
import functools

import jax
import jax.numpy as jnp
from jax.experimental import pallas as pl
from jax.experimental.pallas import tpu as pltpu
from jax.experimental.pallas import tpu_sc as plsc

_EPS = 1e-5

_NC = 2
_NS = 16
_NW = _NC * _NS
_L = 16

_HW = 1024
_ROWS = 64 * 384
_RPW = _ROWS // _NW
_CH = 24
_NCHUNK = _RPW // _CH


def _ln(x, gamma, beta):
    mean = jnp.mean(x, axis=-1, keepdims=True)
    var = jnp.mean((x - mean) ** 2, axis=-1, keepdims=True)
    return (x - mean) * jax.lax.rsqrt(var + _EPS) * gamma + beta


def _scale_body(tv_ref, tt_ref, vt_ref, tg_ref, tb_ref, vg_ref, vb_ref,
                scale_ref):
    B = tv_ref.shape[0]
    V = tt_ref.shape[0]
    idx = tv_ref[:]
    iota = jax.lax.broadcasted_iota(jnp.int32, (B, V), 1)
    oh_t = (iota == idx[:, 0:1]).astype(jnp.float32)
    oh_v = (iota == idx[:, 1:2]).astype(jnp.float32)
    temb = jnp.dot(oh_t, tt_ref[:], preferred_element_type=jnp.float32,
                   precision=jax.lax.Precision.HIGHEST)
    vemb = jnp.dot(oh_v, vt_ref[:], preferred_element_type=jnp.float32,
                   precision=jax.lax.Precision.HIGHEST)
    tln = _ln(temb, tg_ref[:], tb_ref[:])
    vln = _ln(vemb, vg_ref[:], vb_ref[:])
    scale_ref[:] = tln * vln


def _mul_chunk(in_buf, out_buf, scale_vmem, scale_base):
    def row_body(r, _):
        sidx = jnp.full((_L,), scale_base + r, dtype=jnp.int32)
        svec = plsc.load_gather(scale_vmem, [sidx])
        row0 = pl.multiple_of(r * _HW, _HW)
        for v in range(_HW // _L):
            sl = pl.ds(row0 + v * _L, _L)
            out_buf[sl] = in_buf[sl] * svec
        return 0

    jax.lax.fori_loop(0, _CH, row_body, 0)


def _sc_mul_body(scale_hbm, ft_hbm, out_hbm, in0, in1, ob0, ob1, scale_vmem,
                 gsem0, gsem1, ssem0, ssem1):
    wid = jax.lax.axis_index("s") * _NC + jax.lax.axis_index("c")
    base = wid * _RPW

    ins = (in0, in1)
    obs = (ob0, ob1)
    gsems = (gsem0, gsem1)
    ssems = (ssem0, ssem1)

    pltpu.sync_copy(scale_hbm.at[pl.ds(pl.multiple_of(base, _RPW), _RPW)],
                    scale_vmem)

    def rows(i):
        start = pl.multiple_of((base + i * _CH) * _HW, _CH * _HW)
        return pl.ds(start, _CH * _HW)

    def gather(i, s):
        return pltpu.make_async_copy(ft_hbm.at[rows(i)], ins[s], gsems[s])

    def scatter(i, s):
        return pltpu.make_async_copy(obs[s], out_hbm.at[rows(i)], ssems[s])

    gather(0, 0).start()
    gather(1, 1).start()

    def pair_body(g, _):
        for s in range(2):
            i = g * 2 + s
            gather(i, s).wait()

            @pl.when(i >= 2)
            def _():
                scatter(i - 2, s).wait()

            _mul_chunk(ins[s], obs[s], scale_vmem, i * _CH)
            scatter(i, s).start()

            @pl.when(i + 2 < _NCHUNK)
            def _():
                gather(i + 2, s).start()
        return 0

    jax.lax.fori_loop(0, _NCHUNK // 2, pair_body, 0)

    scatter(_NCHUNK - 2, 0).wait()
    scatter(_NCHUNK - 1, 1).wait()


def kernel(ft, taskvar, task_table, var_table, task_gamma, task_beta,
           var_gamma, var_beta):
    B, C, H, W = ft.shape

    scale = pl.pallas_call(
        _scale_body,
        out_shape=jax.ShapeDtypeStruct((B, C), jnp.float32),
    )(taskvar, task_table, var_table,
      task_gamma.reshape(1, C), task_beta.reshape(1, C),
      var_gamma.reshape(1, C), var_beta.reshape(1, C))

    mesh = plsc.VectorSubcoreMesh(core_axis_name="c", subcore_axis_name="s")
    sc_mul = pl.kernel(
        _sc_mul_body,
        out_type=jax.ShapeDtypeStruct((_ROWS * _HW,), jnp.float32),
        mesh=mesh,
        compiler_params=pltpu.CompilerParams(needs_layout_passes=False),
        scratch_types=[
            pltpu.VMEM((_CH * _HW,), jnp.float32),
            pltpu.VMEM((_CH * _HW,), jnp.float32),
            pltpu.VMEM((_CH * _HW,), jnp.float32),
            pltpu.VMEM((_CH * _HW,), jnp.float32),
            pltpu.VMEM((_RPW,), jnp.float32),
            pltpu.SemaphoreType.DMA,
            pltpu.SemaphoreType.DMA,
            pltpu.SemaphoreType.DMA,
            pltpu.SemaphoreType.DMA,
        ],
    )
    out2 = sc_mul(scale.reshape(_ROWS), ft.reshape(_ROWS * _HW))
    return out2.reshape(B, C, H, W)


# speedup vs baseline: 3.3593x; 3.3593x over previous
import jax
import jax.numpy as jnp
from jax.experimental import pallas as pl
from jax.experimental.pallas import tpu as pltpu

_EPS = 1e-5
_K = 8
_LANES = 128


def _ln(x, gamma, beta):
    mean = jnp.mean(x, axis=-1, keepdims=True)
    var = jnp.mean((x - mean) ** 2, axis=-1, keepdims=True)
    return (x - mean) * jax.lax.rsqrt(var + _EPS) * gamma + beta


def _scale_body(tv_ref, tt_ref, vt_ref, tg_ref, tb_ref, vg_ref, vb_ref,
                scale_ref):
    B = tv_ref.shape[0]
    V = tt_ref.shape[0]
    idx = tv_ref[:]
    iota = jax.lax.broadcasted_iota(jnp.int32, (B, V), 1)
    oh_t = (iota == idx[:, 0:1]).astype(jnp.float32)
    oh_v = (iota == idx[:, 1:2]).astype(jnp.float32)
    temb = jnp.dot(oh_t, tt_ref[:], preferred_element_type=jnp.float32,
                   precision=jax.lax.Precision.HIGHEST)
    vemb = jnp.dot(oh_v, vt_ref[:], preferred_element_type=jnp.float32,
                   precision=jax.lax.Precision.HIGHEST)
    tln = _ln(temb, tg_ref[:], tb_ref[:])
    vln = _ln(vemb, vg_ref[:], vb_ref[:])
    scale_ref[:] = jnp.broadcast_to((tln * vln)[:, :, None],
                                    scale_ref.shape)


def _mul_body(scale_ref, ft_ref, out_ref, *rest):
    in_bufs = rest[:_K]
    out_bufs = rest[_K:2 * _K]
    gsems = rest[2 * _K:3 * _K]
    ssems = rest[3 * _K:4 * _K]
    n_groups = pl.num_programs(0)
    N = n_groups * _K
    C, HW = ft_ref.shape[1], ft_ref.shape[2]
    g = pl.program_id(0)

    def gather(i, s):
        return pltpu.make_async_copy(
            ft_ref.at[pl.ds(i, 1)], in_bufs[s], gsems[s])

    def scatter(i, s):
        return pltpu.make_async_copy(
            out_bufs[s], out_ref.at[pl.ds(i, 1)], ssems[s])

    @pl.when(g == 0)
    def _():
        for s in range(_K):
            gather(s, s).start()

    for s in range(_K):
        i = g * _K + s
        gather(i, s).wait()

        @pl.when(g >= 1)
        def _():
            scatter(i - _K, s).wait()

        col = scale_ref[s]
        for v in range(HW // _LANES):
            sl = pl.ds(v * _LANES, _LANES)
            out_bufs[s][0, :, sl] = in_bufs[s][0, :, sl] * col

        scatter(i, s).start()

        @pl.when(g + 1 < n_groups)
        def _():
            gather(i + _K, s).start()

    @pl.when(g == n_groups - 1)
    def _():
        for s in range(_K):
            scatter(N - _K + s, s).wait()


def kernel(ft, taskvar, task_table, var_table, task_gamma, task_beta,
           var_gamma, var_beta):
    B, C, H, W = ft.shape
    HW = H * W

    scale = pl.pallas_call(
        _scale_body,
        out_shape=jax.ShapeDtypeStruct((B, C, _LANES), jnp.float32),
    )(taskvar, task_table, var_table,
      task_gamma.reshape(1, C), task_beta.reshape(1, C),
      var_gamma.reshape(1, C), var_beta.reshape(1, C))

    ft3 = ft.reshape(B, C, HW)
    out3 = pl.pallas_call(
        _mul_body,
        grid=(B // _K,),
        in_specs=[
            pl.BlockSpec((_K, C, _LANES), lambda g: (g, 0, 0)),
            pl.BlockSpec(memory_space=pl.ANY),
        ],
        out_specs=pl.BlockSpec(memory_space=pl.ANY),
        out_shape=jax.ShapeDtypeStruct((B, C, HW), ft.dtype),
        scratch_shapes=(
            [pltpu.VMEM((1, C, HW), jnp.float32) for _ in range(2 * _K)]
            + [pltpu.SemaphoreType.DMA for _ in range(2 * _K)]
        ),
    )(scale, ft3)
    return out3.reshape(B, C, H, W)
